# baseline (speedup 1.0000x reference)
import functools

import jax
import jax.numpy as jnp
from jax.experimental import pallas as pl


_ROW_BLOCK = 1024


def _min_over_rows(x_ref, y_ref, n_rows):
    n = y_ref.shape[1]
    blk = min(_ROW_BLOCK, n_rows)
    y = y_ref[0]

    def step(i, acc):
        xb = x_ref[0, pl.ds(i * blk, blk), :]
        t = jax.lax.dot_general(
            xb, y, (((1,), (1,)), ((), ())),
            preferred_element_type=jnp.float32,
        )
        return jnp.minimum(acc, jnp.min(t, axis=0, keepdims=True))

    acc0 = jnp.full((1, n), jnp.inf, dtype=jnp.float32)
    return jax.lax.fori_loop(0, n_rows // blk, step, acc0)


def _topk_sum(x, k):
    bits = jax.lax.bitcast_convert_type(x, jnp.int32)

    def bs(_, lohi):
        lo, hi = lohi
        mid = lo + (hi - lo + 1) // 2
        cnt = jnp.sum((bits >= mid).astype(jnp.int32))
        take = cnt >= k
        return jnp.where(take, mid, lo), jnp.where(take, hi, mid - 1)

    lo, _ = jax.lax.fori_loop(
        0, 32, bs, (jnp.int32(0), jnp.int32(0x7F000000)))
    v = jax.lax.bitcast_convert_type(lo, jnp.float32)
    sum_gt = jnp.sum(jnp.where(x > v, x, 0.0))
    cnt_gt = jnp.sum((x > v).astype(jnp.float32))
    return sum_gt + (jnp.float32(k) - cnt_gt) * v


def _loss_kernel(xa_ref, ya_ref, xb_ref, yb_ref, norms_ref, out_ref, *, n, k):
    p2 = norms_ref[0, 0:1, :]
    g2 = norms_ref[0, 1:2, :]
    m2 = _min_over_rows(xa_ref, ya_ref, n)
    dist2 = jnp.sqrt(jnp.maximum(m2 + g2, 0.0))
    m1 = _min_over_rows(xb_ref, yb_ref, n)
    dist1 = jnp.sqrt(jnp.maximum(m1 + p2, 0.0))
    inv_n = jnp.float32(1.0 / n)
    loss_cd = (jnp.sum(dist1) + jnp.sum(dist2)) * inv_n
    loss_w = (_topk_sum(dist1, k) + _topk_sum(dist2, k)) * jnp.float32(1.0 / k)
    out_ref[0, 0, :] = jnp.full((128,), loss_cd + 3.0 * loss_w, jnp.float32)


def _augment(x):
    b, n, _ = x.shape
    x2 = jnp.sum(x * x, axis=-1, keepdims=True)
    hi_f = jax.lax.bitcast_convert_type(
        jax.lax.bitcast_convert_type(x2, jnp.int32) & jnp.int32(-65536),
        jnp.float32)
    hi = hi_f.astype(jnp.bfloat16)
    lo = (x2 - hi_f).astype(jnp.bfloat16)
    zpad = jnp.zeros((b, n, 3), jnp.bfloat16)
    return jnp.concatenate(
        [-2.0 * x.astype(jnp.bfloat16), hi, lo, zpad], axis=-1)


def _queries(y):
    b, n, _ = y.shape
    ones = jnp.ones((b, n, 2), jnp.bfloat16)
    zpad = jnp.zeros((b, n, 3), jnp.bfloat16)
    return jnp.concatenate([y.astype(jnp.bfloat16), ones, zpad], axis=-1)


def kernel(pred_pointclouds, gt_pointclouds):
    pred = pred_pointclouds.astype(jnp.float32)
    gt = gt_pointclouds.astype(jnp.float32)
    b, n, _ = pred.shape
    k = int(0.5 * n)

    xa = _augment(pred)
    ya = _queries(gt)
    xb = _augment(gt)
    yb = _queries(pred)
    p2 = jnp.sum(pred * pred, axis=-1)
    g2 = jnp.sum(gt * gt, axis=-1)
    norms = jnp.concatenate(
        [p2[:, None, :], g2[:, None, :],
         jnp.zeros((b, 6, n), jnp.float32)], axis=1)

    spec = pl.BlockSpec((1, n, 8), lambda i: (i, 0, 0))
    losses = pl.pallas_call(
        functools.partial(_loss_kernel, n=n, k=k),
        grid=(b,),
        in_specs=[spec, spec, spec, spec,
                  pl.BlockSpec((1, 8, n), lambda i: (i, 0, 0))],
        out_specs=pl.BlockSpec((1, 1, 128), lambda i: (i, 0, 0)),
        out_shape=jax.ShapeDtypeStruct((b, 1, 128), jnp.float32),
    )(xa, ya, xb, yb, norms)
    return jnp.sum(losses[:, 0, 0]) / b
